# baseline (device time: 18357 ns/iter reference)
import jax
import jax.numpy as jnp
from jax import lax
from jax.experimental import pallas as pl
from jax.experimental.pallas import tpu as pltpu

N_DEV = 4
B, Sq, Skv, Dh = 2, 256, 256, 64
H_PER = 4
D_MODEL = 512
CH = (B * Sq) // N_DEV


def kernel(x, Wq, K_ext, V_ext, Wo):
    my_pos = lax.axis_index("i")
    k = lax.dynamic_slice_in_dim(K_ext, my_pos * H_PER, H_PER, axis=2)
    v = lax.dynamic_slice_in_dim(V_ext, my_pos * H_PER, H_PER, axis=2)
    k = jnp.transpose(k, (0, 2, 1, 3))
    v = jnp.transpose(v, (0, 2, 1, 3))
    x = x.astype(jnp.bfloat16)
    Wq = Wq.astype(jnp.bfloat16)
    k = k.astype(jnp.bfloat16)
    v = v.astype(jnp.bfloat16)
    Wo = Wo.astype(jnp.bfloat16)

    def body(x_ref, wq_ref, k_ref, v_ref, wo_ref, out_ref,
             ctx_ref, part_ref, rs_ref, ag_ref,
             rs_send, rs_recv, ag_send, ag_recv):
        my = lax.axis_index("i")

        barrier_sem = pltpu.get_barrier_semaphore()
        for off in (1, 2, 3):
            pl.semaphore_signal(
                barrier_sem, inc=1,
                device_id=((my + off) % N_DEV,),
                device_id_type=pl.DeviceIdType.MESH,
            )
        pl.semaphore_wait(barrier_sem, 3)

        qi = lax.broadcasted_iota(jnp.int32, (Sq, Skv), 0)
        ki = lax.broadcasted_iota(jnp.int32, (Sq, Skv), 1)
        mask = (jnp.abs(qi - ki) <= 128) | (ki < 32) | (qi < 32)

        for b in range(B):
            q_all = jnp.dot(x_ref[b], wq_ref[...],
                            preferred_element_type=jnp.float32)
            q_all = q_all.astype(jnp.bfloat16)
            for h in range(H_PER):
                q = q_all[:, h * Dh:(h + 1) * Dh]
                kh = k_ref[b, h]
                vh = v_ref[b, h]
                s = lax.dot_general(
                    q, kh, (((1,), (1,)), ((), ())),
                    preferred_element_type=jnp.float32) * 0.125
                s = jnp.where(mask, s, -1e9)
                m = jnp.max(s, axis=1, keepdims=True)
                w = jnp.exp(s - m)
                w = (w / jnp.sum(w, axis=1, keepdims=True)).astype(jnp.bfloat16)
                ctx_ref[b, :, h * Dh:(h + 1) * Dh] = jnp.dot(
                    w, vh, preferred_element_type=jnp.float32).astype(jnp.bfloat16)
            partial = jnp.dot(ctx_ref[b], wo_ref[...],
                              preferred_element_type=jnp.float32)
            part_ref[2 * b] = partial[:CH].astype(jnp.bfloat16)
            part_ref[2 * b + 1] = partial[CH:].astype(jnp.bfloat16)

            for d in (2 * b, 2 * b + 1):
                @pl.when(my != d)
                def _(d=d):
                    pltpu.make_async_remote_copy(
                        src_ref=part_ref.at[d],
                        dst_ref=rs_ref.at[my],
                        send_sem=rs_send.at[d],
                        recv_sem=rs_recv.at[my],
                        device_id=(d,),
                        device_id_type=pl.DeviceIdType.MESH,
                    ).start()

        for s in range(N_DEV):
            @pl.when(my != s)
            def _(s=s):
                pltpu.make_async_remote_copy(
                    src_ref=part_ref.at[s],
                    dst_ref=rs_ref.at[s],
                    send_sem=rs_send.at[s],
                    recv_sem=rs_recv.at[s],
                    device_id=(s,),
                    device_id_type=pl.DeviceIdType.MESH,
                ).wait_recv()

        for c in range(N_DEV):
            @pl.when(my == c)
            def _(c=c):
                red = part_ref[c].astype(jnp.float32)
                for s in range(N_DEV):
                    if s != c:
                        red = red + rs_ref[s].astype(jnp.float32)
                ag_ref[c] = red.astype(jnp.bfloat16)

        for d in range(N_DEV):
            @pl.when(my != d)
            def _(d=d):
                pltpu.make_async_remote_copy(
                    src_ref=ag_ref.at[my],
                    dst_ref=ag_ref.at[my],
                    send_sem=ag_send.at[d],
                    recv_sem=ag_recv.at[my],
                    device_id=(d,),
                    device_id_type=pl.DeviceIdType.MESH,
                ).start()

        for s in range(N_DEV):
            @pl.when(my != s)
            def _(s=s):
                pltpu.make_async_remote_copy(
                    src_ref=ag_ref.at[s],
                    dst_ref=ag_ref.at[s],
                    send_sem=ag_send.at[s],
                    recv_sem=ag_recv.at[s],
                    device_id=(s,),
                    device_id_type=pl.DeviceIdType.MESH,
                ).wait_recv()

        for d in range(N_DEV):
            @pl.when(my != d)
            def _(d=d):
                for sem in (rs_send, ag_send):
                    pltpu.make_async_remote_copy(
                        src_ref=part_ref.at[d],
                        dst_ref=rs_ref.at[d],
                        send_sem=sem.at[d],
                        recv_sem=rs_recv.at[d],
                        device_id=(d,),
                        device_id_type=pl.DeviceIdType.MESH,
                    ).wait_send()

        out_ref[0, 0:CH, :] = ag_ref[0].astype(jnp.float32)
        out_ref[0, CH:Sq, :] = ag_ref[1].astype(jnp.float32)
        out_ref[1, 0:CH, :] = ag_ref[2].astype(jnp.float32)
        out_ref[1, CH:Sq, :] = ag_ref[3].astype(jnp.float32)

    return pl.pallas_call(
        body,
        out_shape=jax.ShapeDtypeStruct((B, Sq, D_MODEL), jnp.float32),
        in_specs=[pl.BlockSpec(memory_space=pltpu.VMEM)] * 5,
        out_specs=pl.BlockSpec(memory_space=pltpu.VMEM),
        scratch_shapes=[
            pltpu.VMEM((B, Sq, H_PER * Dh), jnp.bfloat16),
            pltpu.VMEM((N_DEV, CH, D_MODEL), jnp.bfloat16),
            pltpu.VMEM((N_DEV, CH, D_MODEL), jnp.bfloat16),
            pltpu.VMEM((N_DEV, CH, D_MODEL), jnp.bfloat16),
            pltpu.SemaphoreType.DMA((N_DEV,)),
            pltpu.SemaphoreType.DMA((N_DEV,)),
            pltpu.SemaphoreType.DMA((N_DEV,)),
            pltpu.SemaphoreType.DMA((N_DEV,)),
        ],
        compiler_params=pltpu.CompilerParams(collective_id=0),
    )(x, Wq, k, v, Wo)


# device time: 18169 ns/iter; 1.0103x vs baseline; 1.0103x over previous
import jax
import jax.numpy as jnp
from jax import lax
from jax.experimental import pallas as pl
from jax.experimental.pallas import tpu as pltpu

N_DEV = 4
B, Sq, Skv, Dh = 2, 256, 256, 64
H_PER = 4
D_MODEL = 512
CH = (B * Sq) // N_DEV


def kernel(x, Wq, K_ext, V_ext, Wo):
    my_pos = lax.axis_index("i")
    k = lax.dynamic_slice_in_dim(K_ext, my_pos * H_PER, H_PER, axis=2)
    v = lax.dynamic_slice_in_dim(V_ext, my_pos * H_PER, H_PER, axis=2)
    k = jnp.transpose(k, (0, 2, 1, 3))
    v = jnp.transpose(v, (0, 2, 1, 3))

    def body(x_ref, wq_ref, k_ref, v_ref, wo_ref, out_ref,
             ctx_ref, part_ref, rs_ref, ag_ref,
             rs_send, rs_recv, ag_send, ag_recv):
        my = lax.axis_index("i")

        barrier_sem = pltpu.get_barrier_semaphore()
        for off in (1, 2, 3):
            pl.semaphore_signal(
                barrier_sem, inc=1,
                device_id=((my + off) % N_DEV,),
                device_id_type=pl.DeviceIdType.MESH,
            )
        pl.semaphore_wait(barrier_sem, 3)

        qi = lax.broadcasted_iota(jnp.int32, (Sq, Skv), 0)
        ki = lax.broadcasted_iota(jnp.int32, (Sq, Skv), 1)
        mask = (jnp.abs(qi - ki) <= 128) | (ki < 32) | (qi < 32)

        wq16 = wq_ref[...].astype(jnp.bfloat16)
        wo16 = wo_ref[...].astype(jnp.bfloat16)
        for b in range(B):
            q_all = jnp.dot(x_ref[b].astype(jnp.bfloat16), wq16,
                            preferred_element_type=jnp.float32)
            q_all = q_all.astype(jnp.bfloat16)
            for h in range(H_PER):
                q = q_all[:, h * Dh:(h + 1) * Dh]
                kh = k_ref[b, h].astype(jnp.bfloat16)
                vh = v_ref[b, h].astype(jnp.bfloat16)
                s = lax.dot_general(
                    q, kh, (((1,), (1,)), ((), ())),
                    preferred_element_type=jnp.float32) * 0.125
                s = jnp.where(mask, s, -1e9)
                m = jnp.max(s, axis=1, keepdims=True)
                w = jnp.exp(s - m)
                w = (w / jnp.sum(w, axis=1, keepdims=True)).astype(jnp.bfloat16)
                ctx_ref[b, :, h * Dh:(h + 1) * Dh] = jnp.dot(
                    w, vh, preferred_element_type=jnp.float32).astype(jnp.bfloat16)
            partial = jnp.dot(ctx_ref[b], wo16,
                              preferred_element_type=jnp.float32)
            part_ref[2 * b] = partial[:CH].astype(jnp.bfloat16)
            part_ref[2 * b + 1] = partial[CH:].astype(jnp.bfloat16)

            for d in (2 * b, 2 * b + 1):
                @pl.when(my != d)
                def _(d=d):
                    pltpu.make_async_remote_copy(
                        src_ref=part_ref.at[d],
                        dst_ref=rs_ref.at[my],
                        send_sem=rs_send.at[d],
                        recv_sem=rs_recv.at[my],
                        device_id=(d,),
                        device_id_type=pl.DeviceIdType.MESH,
                    ).start()

        for s in range(N_DEV):
            @pl.when(my != s)
            def _(s=s):
                pltpu.make_async_remote_copy(
                    src_ref=part_ref.at[s],
                    dst_ref=rs_ref.at[s],
                    send_sem=rs_send.at[s],
                    recv_sem=rs_recv.at[s],
                    device_id=(s,),
                    device_id_type=pl.DeviceIdType.MESH,
                ).wait_recv()

        for c in range(N_DEV):
            @pl.when(my == c)
            def _(c=c):
                red = part_ref[c].astype(jnp.float32)
                for s in range(N_DEV):
                    if s != c:
                        red = red + rs_ref[s].astype(jnp.float32)
                ag_ref[c] = red.astype(jnp.bfloat16)

        for d in range(N_DEV):
            @pl.when(my != d)
            def _(d=d):
                pltpu.make_async_remote_copy(
                    src_ref=ag_ref.at[my],
                    dst_ref=ag_ref.at[my],
                    send_sem=ag_send.at[d],
                    recv_sem=ag_recv.at[my],
                    device_id=(d,),
                    device_id_type=pl.DeviceIdType.MESH,
                ).start()

        for s in range(N_DEV):
            @pl.when(my != s)
            def _(s=s):
                pltpu.make_async_remote_copy(
                    src_ref=ag_ref.at[s],
                    dst_ref=ag_ref.at[s],
                    send_sem=ag_send.at[s],
                    recv_sem=ag_recv.at[s],
                    device_id=(s,),
                    device_id_type=pl.DeviceIdType.MESH,
                ).wait_recv()

        for d in range(N_DEV):
            @pl.when(my != d)
            def _(d=d):
                for sem in (rs_send, ag_send):
                    pltpu.make_async_remote_copy(
                        src_ref=part_ref.at[d],
                        dst_ref=rs_ref.at[d],
                        send_sem=sem.at[d],
                        recv_sem=rs_recv.at[d],
                        device_id=(d,),
                        device_id_type=pl.DeviceIdType.MESH,
                    ).wait_send()

        out_ref[0, 0:CH, :] = ag_ref[0].astype(jnp.float32)
        out_ref[0, CH:Sq, :] = ag_ref[1].astype(jnp.float32)
        out_ref[1, 0:CH, :] = ag_ref[2].astype(jnp.float32)
        out_ref[1, CH:Sq, :] = ag_ref[3].astype(jnp.float32)

    return pl.pallas_call(
        body,
        out_shape=jax.ShapeDtypeStruct((B, Sq, D_MODEL), jnp.float32),
        in_specs=[pl.BlockSpec(memory_space=pltpu.VMEM)] * 5,
        out_specs=pl.BlockSpec(memory_space=pltpu.VMEM),
        scratch_shapes=[
            pltpu.VMEM((B, Sq, H_PER * Dh), jnp.bfloat16),
            pltpu.VMEM((N_DEV, CH, D_MODEL), jnp.bfloat16),
            pltpu.VMEM((N_DEV, CH, D_MODEL), jnp.bfloat16),
            pltpu.VMEM((N_DEV, CH, D_MODEL), jnp.bfloat16),
            pltpu.SemaphoreType.DMA((N_DEV,)),
            pltpu.SemaphoreType.DMA((N_DEV,)),
            pltpu.SemaphoreType.DMA((N_DEV,)),
            pltpu.SemaphoreType.DMA((N_DEV,)),
        ],
        compiler_params=pltpu.CompilerParams(collective_id=0),
    )(x, Wq, k, v, Wo)


# device time: 17966 ns/iter; 1.0218x vs baseline; 1.0113x over previous
import jax
import jax.numpy as jnp
from jax import lax
from jax.experimental import pallas as pl
from jax.experimental.pallas import tpu as pltpu

N_DEV = 4
B, Sq, Skv, Dh = 2, 256, 256, 64
H_PER = 4
D_MODEL = 512
CH = (B * Sq) // N_DEV


def kernel(x, Wq, K_ext, V_ext, Wo):
    my_pos = lax.axis_index("i")
    k = lax.dynamic_slice_in_dim(K_ext, my_pos * H_PER, H_PER, axis=2)
    v = lax.dynamic_slice_in_dim(V_ext, my_pos * H_PER, H_PER, axis=2)
    k = jnp.transpose(k, (0, 2, 1, 3))
    v = jnp.transpose(v, (0, 2, 1, 3))

    def body(x_ref, wq_ref, k_ref, v_ref, wo_ref, out_ref,
             ctx_ref, part_ref, rs_ref, ag_ref,
             rs_send, rs_recv, ag_send, ag_recv):
        my = lax.axis_index("i")

        barrier_sem = pltpu.get_barrier_semaphore()
        for off in (1, 2, 3):
            pl.semaphore_signal(
                barrier_sem, inc=1,
                device_id=((my + off) % N_DEV,),
                device_id_type=pl.DeviceIdType.MESH,
            )
        pl.semaphore_wait(barrier_sem, 3)

        qi = lax.broadcasted_iota(jnp.int32, (Sq, Skv), 0)
        ki = lax.broadcasted_iota(jnp.int32, (Sq, Skv), 1)
        mask = (jnp.abs(qi - ki) <= 128) | (ki < 32) | (qi < 32)

        wq16 = wq_ref[...].astype(jnp.bfloat16)
        wo16 = wo_ref[...].astype(jnp.bfloat16)
        for b in range(B):
            q_all = jnp.dot(x_ref[b].astype(jnp.bfloat16), wq16,
                            preferred_element_type=jnp.float32)
            q_all = q_all.astype(jnp.bfloat16)
            for h in range(H_PER):
                q = q_all[:, h * Dh:(h + 1) * Dh]
                kh = k_ref[b, h].astype(jnp.bfloat16)
                vh = v_ref[b, h].astype(jnp.bfloat16)
                s = lax.dot_general(
                    q, kh, (((1,), (1,)), ((), ())),
                    preferred_element_type=jnp.float32) * 0.125
                w = jnp.exp(jnp.where(mask, s, -1e9))
                w = (w / jnp.sum(w, axis=1, keepdims=True)).astype(jnp.bfloat16)
                ctx_ref[b, :, h * Dh:(h + 1) * Dh] = jnp.dot(
                    w, vh, preferred_element_type=jnp.float32).astype(jnp.bfloat16)
            partial = jnp.dot(ctx_ref[b], wo16,
                              preferred_element_type=jnp.float32)
            part_ref[2 * b] = partial[:CH].astype(jnp.bfloat16)
            part_ref[2 * b + 1] = partial[CH:].astype(jnp.bfloat16)

            for d in (2 * b, 2 * b + 1):
                @pl.when(my != d)
                def _(d=d):
                    pltpu.make_async_remote_copy(
                        src_ref=part_ref.at[d],
                        dst_ref=rs_ref.at[my],
                        send_sem=rs_send.at[d],
                        recv_sem=rs_recv.at[my],
                        device_id=(d,),
                        device_id_type=pl.DeviceIdType.MESH,
                    ).start()

        for s in range(N_DEV):
            @pl.when(my != s)
            def _(s=s):
                pltpu.make_async_remote_copy(
                    src_ref=part_ref.at[s],
                    dst_ref=rs_ref.at[s],
                    send_sem=rs_send.at[s],
                    recv_sem=rs_recv.at[s],
                    device_id=(s,),
                    device_id_type=pl.DeviceIdType.MESH,
                ).wait_recv()

        for c in range(N_DEV):
            @pl.when(my == c)
            def _(c=c):
                red = part_ref[c].astype(jnp.float32)
                for s in range(N_DEV):
                    if s != c:
                        red = red + rs_ref[s].astype(jnp.float32)
                ag_ref[c] = red.astype(jnp.bfloat16)

        for d in range(N_DEV):
            @pl.when(my != d)
            def _(d=d):
                pltpu.make_async_remote_copy(
                    src_ref=ag_ref.at[my],
                    dst_ref=ag_ref.at[my],
                    send_sem=ag_send.at[d],
                    recv_sem=ag_recv.at[my],
                    device_id=(d,),
                    device_id_type=pl.DeviceIdType.MESH,
                ).start()

        for s in range(N_DEV):
            @pl.when(my != s)
            def _(s=s):
                pltpu.make_async_remote_copy(
                    src_ref=ag_ref.at[s],
                    dst_ref=ag_ref.at[s],
                    send_sem=ag_send.at[s],
                    recv_sem=ag_recv.at[s],
                    device_id=(s,),
                    device_id_type=pl.DeviceIdType.MESH,
                ).wait_recv()
            out_ref[s // 2, (s % 2) * CH:(s % 2) * CH + CH, :] = (
                ag_ref[s].astype(jnp.float32))

        for d in range(N_DEV):
            @pl.when(my != d)
            def _(d=d):
                for sem in (rs_send, ag_send):
                    pltpu.make_async_remote_copy(
                        src_ref=part_ref.at[d],
                        dst_ref=rs_ref.at[d],
                        send_sem=sem.at[d],
                        recv_sem=rs_recv.at[d],
                        device_id=(d,),
                        device_id_type=pl.DeviceIdType.MESH,
                    ).wait_send()

    return pl.pallas_call(
        body,
        out_shape=jax.ShapeDtypeStruct((B, Sq, D_MODEL), jnp.float32),
        in_specs=[pl.BlockSpec(memory_space=pltpu.VMEM)] * 5,
        out_specs=pl.BlockSpec(memory_space=pltpu.VMEM),
        scratch_shapes=[
            pltpu.VMEM((B, Sq, H_PER * Dh), jnp.bfloat16),
            pltpu.VMEM((N_DEV, CH, D_MODEL), jnp.bfloat16),
            pltpu.VMEM((N_DEV, CH, D_MODEL), jnp.bfloat16),
            pltpu.VMEM((N_DEV, CH, D_MODEL), jnp.bfloat16),
            pltpu.SemaphoreType.DMA((N_DEV,)),
            pltpu.SemaphoreType.DMA((N_DEV,)),
            pltpu.SemaphoreType.DMA((N_DEV,)),
            pltpu.SemaphoreType.DMA((N_DEV,)),
        ],
        compiler_params=pltpu.CompilerParams(collective_id=0),
    )(x, Wq, k, v, Wo)


# device time: 16578 ns/iter; 1.1073x vs baseline; 1.0837x over previous
import jax
import jax.numpy as jnp
from jax import lax
from jax.experimental import pallas as pl
from jax.experimental.pallas import tpu as pltpu

N_DEV = 4
B, Sq, Skv, Dh = 2, 256, 256, 64
H_PER = 4
D_MODEL = 512
CH = (B * Sq) // N_DEV
HALF = CH // 2


def kernel(x, Wq, K_ext, V_ext, Wo):
    my_pos = lax.axis_index("i")
    k = lax.dynamic_slice_in_dim(K_ext, my_pos * H_PER, H_PER, axis=2)
    v = lax.dynamic_slice_in_dim(V_ext, my_pos * H_PER, H_PER, axis=2)
    k = jnp.transpose(k, (0, 2, 1, 3))
    v = jnp.transpose(v, (0, 2, 1, 3))

    def body(x_ref, wq_ref, k_ref, v_ref, wo_ref, out_ref,
             ctx_ref, part_ref, rs_ref, ag_ref,
             rs_send, rs_recv, ag_send, ag_recv):
        my = lax.axis_index("i")

        barrier_sem = pltpu.get_barrier_semaphore()
        for off in (1, 2, 3):
            pl.semaphore_signal(
                barrier_sem, inc=1,
                device_id=((my + off) % N_DEV,),
                device_id_type=pl.DeviceIdType.MESH,
            )
        pl.semaphore_wait(barrier_sem, 3)

        qi = lax.broadcasted_iota(jnp.int32, (Sq, Skv), 0)
        ki = lax.broadcasted_iota(jnp.int32, (Sq, Skv), 1)
        mask = (jnp.abs(qi - ki) <= 128) | (ki < 32) | (qi < 32)

        wq16 = wq_ref[...].astype(jnp.bfloat16)
        wo16 = wo_ref[...].astype(jnp.bfloat16)
        for b in range(B):
            q_all = jnp.dot(x_ref[b].astype(jnp.bfloat16), wq16,
                            preferred_element_type=jnp.float32)
            q_all = q_all.astype(jnp.bfloat16)
            for h in range(H_PER):
                q = q_all[:, h * Dh:(h + 1) * Dh]
                kh = k_ref[b, h].astype(jnp.bfloat16)
                vh = v_ref[b, h].astype(jnp.bfloat16)
                s = lax.dot_general(
                    q, kh, (((1,), (1,)), ((), ())),
                    preferred_element_type=jnp.float32) * 0.125
                w = jnp.exp(jnp.where(mask, s, -1e9))
                w = (w / jnp.sum(w, axis=1, keepdims=True)).astype(jnp.bfloat16)
                ctx_ref[b, :, h * Dh:(h + 1) * Dh] = jnp.dot(
                    w, vh, preferred_element_type=jnp.float32).astype(jnp.bfloat16)
            partial = jnp.dot(ctx_ref[b], wo16,
                              preferred_element_type=jnp.float32)
            part_ref[2 * b] = partial[:CH].astype(jnp.bfloat16)
            part_ref[2 * b + 1] = partial[CH:].astype(jnp.bfloat16)

            for d in (2 * b, 2 * b + 1):
                for j in range(2):
                    @pl.when(my != d)
                    def _(d=d, j=j):
                        pltpu.make_async_remote_copy(
                            src_ref=part_ref.at[d, pl.ds(j * HALF, HALF)],
                            dst_ref=rs_ref.at[my, pl.ds(j * HALF, HALF)],
                            send_sem=rs_send.at[j, d],
                            recv_sem=rs_recv.at[j, my],
                            device_id=(d,),
                            device_id_type=pl.DeviceIdType.MESH,
                        ).start()

        for j in range(2):
            for s in range(N_DEV):
                @pl.when(my != s)
                def _(s=s, j=j):
                    pltpu.make_async_remote_copy(
                        src_ref=part_ref.at[s, pl.ds(j * HALF, HALF)],
                        dst_ref=rs_ref.at[s, pl.ds(j * HALF, HALF)],
                        send_sem=rs_send.at[j, s],
                        recv_sem=rs_recv.at[j, s],
                        device_id=(s,),
                        device_id_type=pl.DeviceIdType.MESH,
                    ).wait_recv()

            js = slice(j * HALF, (j + 1) * HALF)
            for c in range(N_DEV):
                @pl.when(my == c)
                def _(c=c, js=js):
                    red = part_ref[c, js].astype(jnp.float32)
                    for s in range(N_DEV):
                        if s != c:
                            red = red + rs_ref[s, js].astype(jnp.float32)
                    ag_ref[c, js] = red.astype(jnp.bfloat16)

            for d in range(N_DEV):
                @pl.when(my != d)
                def _(d=d, j=j):
                    pltpu.make_async_remote_copy(
                        src_ref=ag_ref.at[my, pl.ds(j * HALF, HALF)],
                        dst_ref=ag_ref.at[my, pl.ds(j * HALF, HALF)],
                        send_sem=ag_send.at[j, d],
                        recv_sem=ag_recv.at[j, my],
                        device_id=(d,),
                        device_id_type=pl.DeviceIdType.MESH,
                    ).start()

        for s in range(N_DEV):
            for j in range(2):
                @pl.when(my != s)
                def _(s=s, j=j):
                    pltpu.make_async_remote_copy(
                        src_ref=ag_ref.at[s, pl.ds(j * HALF, HALF)],
                        dst_ref=ag_ref.at[s, pl.ds(j * HALF, HALF)],
                        send_sem=ag_send.at[j, s],
                        recv_sem=ag_recv.at[j, s],
                        device_id=(s,),
                        device_id_type=pl.DeviceIdType.MESH,
                    ).wait_recv()
            out_ref[s // 2, (s % 2) * CH:(s % 2) * CH + CH, :] = (
                ag_ref[s].astype(jnp.float32))

        for d in range(N_DEV):
            for j in range(2):
                @pl.when(my != d)
                def _(d=d, j=j):
                    for sem in (rs_send, ag_send):
                        pltpu.make_async_remote_copy(
                            src_ref=part_ref.at[d, pl.ds(j * HALF, HALF)],
                            dst_ref=rs_ref.at[d, pl.ds(j * HALF, HALF)],
                            send_sem=sem.at[j, d],
                            recv_sem=rs_recv.at[j, d],
                            device_id=(d,),
                            device_id_type=pl.DeviceIdType.MESH,
                        ).wait_send()

    return pl.pallas_call(
        body,
        out_shape=jax.ShapeDtypeStruct((B, Sq, D_MODEL), jnp.float32),
        in_specs=[pl.BlockSpec(memory_space=pltpu.VMEM)] * 5,
        out_specs=pl.BlockSpec(memory_space=pltpu.VMEM),
        scratch_shapes=[
            pltpu.VMEM((B, Sq, H_PER * Dh), jnp.bfloat16),
            pltpu.VMEM((N_DEV, CH, D_MODEL), jnp.bfloat16),
            pltpu.VMEM((N_DEV, CH, D_MODEL), jnp.bfloat16),
            pltpu.VMEM((N_DEV, CH, D_MODEL), jnp.bfloat16),
            pltpu.SemaphoreType.DMA((2, N_DEV)),
            pltpu.SemaphoreType.DMA((2, N_DEV)),
            pltpu.SemaphoreType.DMA((2, N_DEV)),
            pltpu.SemaphoreType.DMA((2, N_DEV)),
        ],
        compiler_params=pltpu.CompilerParams(collective_id=0),
    )(x, Wq, k, v, Wo)
